# Initial kernel scaffold; baseline (speedup 1.0000x reference)
import jax
import jax.numpy as jnp
from jax.experimental import pallas as pl


def kernel(encoder_hidden, current_inputs, edge_index, edge_weight, W_ru_0, b_ru_0, W_c_0, b_c_0, W_ru_1, b_ru_1, W_c_1, b_c_1, W_out, b_out):
    raise NotImplementedError("write your pallas kernel here")


# speedup vs baseline: 2.5033x; 2.5033x over previous
import functools

import jax
import jax.numpy as jnp
import numpy as np
from jax import lax
from jax.experimental import pallas as pl
from jax.experimental.pallas import tpu as pltpu
from jax.experimental.pallas import tpu_sc as plsc

H = 128
N = 10000
L = 2
T = 12
P = 12
E = 320000
OUT = 1

NC = 2
NS = 16
NW = NC * NS
_SIZES_SC = [10080] * 5 + [9968] * 10 + [9920]
SIZES = np.array(_SIZES_SC + _SIZES_SC, np.int64)
STARTS = np.concatenate([[0], np.cumsum(SIZES)[:-1]])
SLAB = 10080
CK = 80
GB = 6
NGRP = SLAB // (CK * GB)
FCH = H // 16
NACC = N + 8
NFB = 128
RPT = 624
TAIL = NACC - RPT * NS


def _build_spmm():
    mesh = plsc.VectorSubcoreMesh(
        core_axis_name="c", subcore_axis_name="s", num_cores=NC, num_subcores=NS
    )

    @functools.partial(
        pl.kernel,
        out_type=jax.ShapeDtypeStruct((NC, N, H), jnp.float32),
        mesh=mesh,
        scratch_types=[
            pltpu.VMEM((GB, CK), jnp.int32),
            pltpu.VMEM((GB, CK), jnp.int32),
            pltpu.VMEM((GB, CK), jnp.float32),
            pltpu.VMEM((NFB, CK), jnp.int32),
            pltpu.VMEM((CK, H), jnp.float32),
            pltpu.VMEM((CK, H), jnp.float32),
            pltpu.VMEM_SHARED((NACC, H), jnp.float32),
            pltpu.SemaphoreType.DMA,
        ],
    )
    def spmm(x_ref, src_ref, dst_ref, ew_ref, fid_ref, z_ref, out_ref,
             src_v, dst_v, ew_v, fid_v, rows_v, fval_v, acc, sem):
        c = lax.axis_index("c")
        s = lax.axis_index("s")
        wid = c * NS + s
        pltpu.sync_copy(fid_ref.at[wid], fid_v)
        pltpu.sync_copy(z_ref.at[pl.ds(0, CK)], fval_v)
        pltpu.sync_copy(z_ref.at[pl.ds(s * RPT, RPT)], acc.at[pl.ds(s * RPT, RPT)])

        @pl.when(s == NS - 1)
        def _zero_tail():
            pltpu.sync_copy(z_ref.at[pl.ds(RPT * NS, TAIL)],
                            acc.at[pl.ds(RPT * NS, TAIL)])

        plsc.subcore_barrier()

        def edge_group_body(q, carry):
            gj, gcarry = carry
            dvec = dst_v[gj, pl.ds(q * 16, 16)]
            wvec = ew_v[gj, pl.ds(q * 16, 16)]
            for l in range(16):
                prev_d, fcnt, bat = gcarry
                d = dvec[l]
                new = d != prev_d

                def adv(opers):
                    fcnt, bat = opers
                    fcnt = fcnt + 1

                    def do_stream(o):
                        fcnt, bat = o
                        pltpu.sync_copy(fval_v, acc.at[fid_v.at[bat]], add=True)
                        return jnp.int32(0), bat + 1

                    return lax.cond(fcnt == CK, do_stream, lambda o: o,
                                    (fcnt, bat))

                fcnt, bat = lax.cond(new, adv, lambda o: o, (fcnt, bat))
                keep = jnp.where(new, 0.0, 1.0)
                j = q * 16 + l
                for f in range(FCH):
                    sl = pl.ds(f * 16, 16)
                    m = rows_v[j, sl] * wvec[l]
                    fval_v[fcnt, sl] = fval_v[fcnt, sl] * keep + m
                gcarry = (d, fcnt, bat)
            return (gj, gcarry)

        def chunk_body(gj, gcarry):
            pltpu.async_copy(x_ref.at[src_v.at[gj]], rows_v, sem).wait()
            _, gcarry = lax.fori_loop(0, CK // 16, edge_group_body, (gj, gcarry))
            return gcarry

        def group_body(gi, gcarry):
            pltpu.sync_copy(src_ref.at[wid].at[gi], src_v)
            pltpu.sync_copy(dst_ref.at[wid].at[gi], dst_v)
            pltpu.sync_copy(ew_ref.at[wid].at[gi], ew_v)
            return lax.fori_loop(0, GB, chunk_body, gcarry)

        init = (jnp.int32(-1), jnp.int32(0), jnp.int32(0))
        _, fcnt, bat = lax.fori_loop(0, NGRP, group_body, init)
        pltpu.sync_copy(fval_v, acc.at[fid_v.at[bat]], add=True)
        plsc.subcore_barrier()
        pltpu.sync_copy(acc.at[pl.ds(s * RPT, RPT)],
                        out_ref.at[c].at[pl.ds(s * RPT, RPT)])

        @pl.when(s == NS - 1)
        def _copy_tail():
            pltpu.sync_copy(acc.at[pl.ds(RPT * NS, N - RPT * NS)],
                            out_ref.at[c].at[pl.ds(RPT * NS, N - RPT * NS)])

    return spmm


_SPMM = None


def _spmm_fn():
    global _SPMM
    if _SPMM is None:
        _SPMM = _build_spmm()
    return _SPMM


BN = 2000


def _gate_body(ax_ref, ah_ref, h_ref, w_ref, b_ref, rh_ref, u_ref):
    ax = ax_ref[0] + ax_ref[1]
    ah = ah_ref[0] + ah_ref[1]
    agg = jnp.concatenate([ax, ah], axis=1)
    ru = jax.nn.sigmoid(
        jnp.dot(agg, w_ref[...], preferred_element_type=jnp.float32) + b_ref[...]
    )
    rh_ref[...] = ru[:, :H] * h_ref[...]
    u_ref[...] = ru[:, H:]


def _tc_gate(aggx, aggh, h, W_ru, b_ru):
    grid = (N // BN,)
    return pl.pallas_call(
        _gate_body,
        grid=grid,
        in_specs=[
            pl.BlockSpec((NC, BN, H), lambda i: (0, i, 0)),
            pl.BlockSpec((NC, BN, H), lambda i: (0, i, 0)),
            pl.BlockSpec((BN, H), lambda i: (i, 0)),
            pl.BlockSpec((2 * H, 2 * H), lambda i: (0, 0)),
            pl.BlockSpec((1, 2 * H), lambda i: (0, 0)),
        ],
        out_specs=[
            pl.BlockSpec((BN, H), lambda i: (i, 0)),
            pl.BlockSpec((BN, H), lambda i: (i, 0)),
        ],
        out_shape=[
            jax.ShapeDtypeStruct((N, H), jnp.float32),
            jax.ShapeDtypeStruct((N, H), jnp.float32),
        ],
    )(aggx, aggh, h, W_ru, b_ru.reshape(1, 2 * H))


def _update_body(ax_ref, arh_ref, h_ref, u_ref, w_ref, b_ref, hn_ref):
    ax = ax_ref[0] + ax_ref[1]
    arh = arh_ref[0] + arh_ref[1]
    agg = jnp.concatenate([ax, arh], axis=1)
    c = jnp.tanh(
        jnp.dot(agg, w_ref[...], preferred_element_type=jnp.float32) + b_ref[...]
    )
    u = u_ref[...]
    hn_ref[...] = u * h_ref[...] + (1.0 - u) * c


def _tc_update(aggx, aggrh, h, u, W_c, b_c):
    grid = (N // BN,)
    return pl.pallas_call(
        _update_body,
        grid=grid,
        in_specs=[
            pl.BlockSpec((NC, BN, H), lambda i: (0, i, 0)),
            pl.BlockSpec((NC, BN, H), lambda i: (0, i, 0)),
            pl.BlockSpec((BN, H), lambda i: (i, 0)),
            pl.BlockSpec((BN, H), lambda i: (i, 0)),
            pl.BlockSpec((2 * H, H), lambda i: (0, 0)),
            pl.BlockSpec((1, H), lambda i: (0, 0)),
        ],
        out_specs=pl.BlockSpec((BN, H), lambda i: (i, 0)),
        out_shape=jax.ShapeDtypeStruct((N, H), jnp.float32),
    )(aggx, aggrh, h, u, W_c, b_c.reshape(1, H))


def _update_out_body(ax_ref, arh_ref, h_ref, u_ref, w_ref, b_ref,
                     wo_ref, bo_ref, hn_ref, o_ref):
    ax = ax_ref[0] + ax_ref[1]
    arh = arh_ref[0] + arh_ref[1]
    agg = jnp.concatenate([ax, arh], axis=1)
    c = jnp.tanh(
        jnp.dot(agg, w_ref[...], preferred_element_type=jnp.float32) + b_ref[...]
    )
    u = u_ref[...]
    hn = u * h_ref[...] + (1.0 - u) * c
    hn_ref[...] = hn
    o_ref[...] = jnp.dot(hn, wo_ref[...], preferred_element_type=jnp.float32) + bo_ref[...]


def _tc_update_out(aggx, aggrh, h, u, W_c, b_c, W_out, b_out):
    grid = (N // BN,)
    return pl.pallas_call(
        _update_out_body,
        grid=grid,
        in_specs=[
            pl.BlockSpec((NC, BN, H), lambda i: (0, i, 0)),
            pl.BlockSpec((NC, BN, H), lambda i: (0, i, 0)),
            pl.BlockSpec((BN, H), lambda i: (i, 0)),
            pl.BlockSpec((BN, H), lambda i: (i, 0)),
            pl.BlockSpec((2 * H, H), lambda i: (0, 0)),
            pl.BlockSpec((1, H), lambda i: (0, 0)),
            pl.BlockSpec((H, OUT), lambda i: (0, 0)),
            pl.BlockSpec((1, OUT), lambda i: (0, 0)),
        ],
        out_specs=[
            pl.BlockSpec((BN, H), lambda i: (i, 0)),
            pl.BlockSpec((BN, OUT), lambda i: (i, 0)),
        ],
        out_shape=[
            jax.ShapeDtypeStruct((N, H), jnp.float32),
            jax.ShapeDtypeStruct((N, OUT), jnp.float32),
        ],
    )(aggx, aggrh, h, u, W_c, b_c.reshape(1, H), W_out, b_out.reshape(1, OUT))


def _prep_edges(edge_index, edge_weight):
    src = edge_index[0]
    dst = edge_index[1]
    order = jnp.argsort(dst, stable=True)
    src_s = src[order]
    dst_s = dst[order]
    ew_s = edge_weight[order]

    starts = jnp.asarray(STARTS, jnp.int32)
    sizes = jnp.asarray(SIZES, jnp.int32)
    i = jnp.arange(SLAB, dtype=jnp.int32)
    posmat = starts[:, None] + jnp.minimum(i[None, :], sizes[:, None] - 1)
    valid = i[None, :] < sizes[:, None]
    dst_slab = jnp.where(valid, dst_s[posmat], N).astype(jnp.int32)
    src_slab = jnp.where(valid, src_s[posmat], 0).astype(jnp.int32)
    ew_slab = jnp.where(valid, ew_s[posmat], 0.0)

    prev = jnp.concatenate(
        [jnp.full((NW, 1), -1, jnp.int32), dst_slab[:, :-1]], axis=1)
    segidx = jnp.cumsum((dst_slab != prev).astype(jnp.int32), axis=1)
    fids = jnp.full((NW, NFB * CK), N, jnp.int32)
    fids = fids.at[jnp.arange(NW)[:, None], segidx].set(dst_slab)

    src3 = src_slab.reshape(NW, NGRP, GB, CK)
    dst3 = dst_slab.reshape(NW, NGRP, GB, CK)
    ew3 = ew_slab.reshape(NW, NGRP, GB, CK)
    fid3 = fids.reshape(NW, NFB, CK)
    return src3, dst3, ew3, fid3


def kernel(encoder_hidden, current_inputs, edge_index, edge_weight,
           W_ru_0, b_ru_0, W_c_0, b_c_0,
           W_ru_1, b_ru_1, W_c_1, b_c_1,
           W_out, b_out):
    src3, dst3, ew3, fid3 = _prep_edges(edge_index, edge_weight)
    zeros = jnp.zeros((NACC, H), jnp.float32)

    spmm_k = _spmm_fn()

    def spmm(x):
        return spmm_k(x, src3, dst3, ew3, fid3, zeros)

    h0 = encoder_hidden[0]
    h1 = encoder_hidden[1]
    x0 = current_inputs[:, -1, :]

    aggx0 = spmm(x0)
    aggh0 = spmm(h0)
    aggh1 = spmm(h1)

    outputs = []
    for p in range(P):
        rh0, u0 = _tc_gate(aggx0, aggh0, h0, W_ru_0, b_ru_0)
        aggrh0 = spmm(rh0)
        h0 = _tc_update(aggx0, aggrh0, h0, u0, W_c_0, b_c_0)
        aggx1 = spmm(h0)
        rh1, u1 = _tc_gate(aggx1, aggh1, h1, W_ru_1, b_ru_1)
        aggrh1 = spmm(rh1)
        h1, out_p = _tc_update_out(aggx1, aggrh1, h1, u1, W_c_1, b_c_1, W_out, b_out)
        outputs.append(out_p)
        if p + 1 < P:
            aggh0 = aggx1
            aggx0 = spmm(h1)
            aggh1 = aggx0

    out = jnp.stack(outputs, axis=1)
    return out, jnp.stack([h0, h1], axis=0)
